# Initial kernel scaffold; baseline (speedup 1.0000x reference)
import jax
import jax.numpy as jnp
from jax.experimental import pallas as pl


def kernel(x, W, db_features, db_labels):
    raise NotImplementedError("write your pallas kernel here")


# speedup vs baseline: 4.2085x; 4.2085x over previous
import functools

import jax
import jax.numpy as jnp
from jax import lax
from jax.experimental import pallas as pl
from jax.experimental.pallas import tpu as pltpu
from jax.experimental.pallas import tpu_sc as plsc


def _pick_chunk(n: int, cap: int = 2048) -> int:
    for c in range(min(n, cap), 7, -1):
        if n % c == 0 and c % 8 == 0:
            return c
    return n


def _topk_body(n_chunk, x_ref, w_ref, db_ref, val_ref, idx_ref, qn_ref):
    i = pl.program_id(0)

    @pl.when(i == 0)
    def _init():
        feats = jnp.dot(x_ref[...], w_ref[...],
                        preferred_element_type=jnp.float32)
        qnorm = jnp.sqrt(jnp.sum(feats * feats, axis=1, keepdims=True))
        qn_ref[...] = feats / (qnorm + 1e-8)
        val_ref[...] = jnp.full(val_ref.shape, -jnp.inf, jnp.float32)
        idx_ref[...] = jnp.zeros(idx_ref.shape, jnp.int32)

    db = db_ref[...]
    dnorm = jnp.sqrt(jnp.sum(db * db, axis=1, keepdims=True))
    dn = db / (dnorm + 1e-8)
    s = lax.dot_general(dn, qn_ref[...], (((1,), (1,)), ((), ())),
                        preferred_element_type=jnp.float32)
    m = jnp.max(s, axis=0)
    rows = lax.broadcasted_iota(jnp.int32, s.shape, 0)
    cand = jnp.min(jnp.where(s == m[None, :], rows, n_chunk), axis=0)
    gidx = cand + i * n_chunk
    better = m > val_ref[...]
    val_ref[...] = jnp.where(better, m, val_ref[...])
    idx_ref[...] = jnp.where(better, gidx, idx_ref[...])


def _topk_call(x, W, db):
    b, d = x.shape
    n = db.shape[0]
    chunk = _pick_chunk(n)
    nsteps = n // chunk
    return pl.pallas_call(
        functools.partial(_topk_body, chunk),
        grid=(nsteps,),
        in_specs=[
            pl.BlockSpec((b, d), lambda i: (0, 0)),
            pl.BlockSpec((d, d), lambda i: (0, 0)),
            pl.BlockSpec((chunk, d), lambda i: (i, 0)),
        ],
        out_specs=[
            pl.BlockSpec((b,), lambda i: (0,)),
            pl.BlockSpec((b,), lambda i: (0,)),
        ],
        out_shape=[
            jax.ShapeDtypeStruct((b,), jnp.float32),
            jax.ShapeDtypeStruct((b,), jnp.int32),
        ],
        scratch_shapes=[pltpu.VMEM((b, d), jnp.float32)],
    )(x, W, db)


def _label_gather(labels, idx):
    b = idx.shape[0]
    info = plsc.get_sparse_core_info()
    nw = info.num_cores * info.num_subcores
    bpw = b // nw
    mesh = plsc.VectorSubcoreMesh(core_axis_name="c", subcore_axis_name="s")

    @functools.partial(
        pl.kernel,
        mesh=mesh,
        out_type=jax.ShapeDtypeStruct((b,), jnp.int32),
        scratch_types=[
            pltpu.VMEM((bpw,), jnp.int32),
            pltpu.VMEM((bpw,), jnp.int32),
            pltpu.SemaphoreType.DMA,
        ],
    )
    def k(labels_hbm, idx_hbm, out_hbm, idx_v, vals_v, sem):
        wid = lax.axis_index("s") * info.num_cores + lax.axis_index("c")
        base = wid * bpw
        pltpu.sync_copy(idx_hbm.at[pl.ds(base, bpw)], idx_v)
        pltpu.async_copy(labels_hbm.at[idx_v], vals_v, sem).wait()
        pltpu.sync_copy(vals_v, out_hbm.at[pl.ds(base, bpw)])

    return k(labels, idx)


def kernel(x, W, db_features, db_labels):
    top_vals, top_idx = _topk_call(x, W, db_features)
    pred = _label_gather(db_labels, top_idx)
    return top_vals, top_idx, pred


# speedup vs baseline: 5.5322x; 1.3145x over previous
import functools

import jax
import jax.numpy as jnp
from jax import lax
from jax.experimental import pallas as pl
from jax.experimental.pallas import tpu as pltpu
from jax.experimental.pallas import tpu_sc as plsc


def _pick_chunk(n: int, cap: int = 2048) -> int:
    for c in range(min(n, cap), 7, -1):
        if n % c == 0 and c % 8 == 0:
            return c
    return n


def _topk_body(n_chunk, x_ref, w_ref, db_ref, val_ref, idx_ref, qn_ref):
    i = pl.program_id(0)

    @pl.when(i == 0)
    def _init():
        feats = jnp.dot(x_ref[...], w_ref[...],
                        preferred_element_type=jnp.float32)
        qnorm = jnp.sqrt(jnp.sum(feats * feats, axis=1, keepdims=True))
        qn_ref[...] = feats / (qnorm + 1e-8)
        val_ref[...] = jnp.full(val_ref.shape, -jnp.inf, jnp.float32)
        idx_ref[...] = jnp.zeros(idx_ref.shape, jnp.int32)

    db = db_ref[...]
    dnorm = jnp.sqrt(jnp.sum(db * db, axis=1, keepdims=True))
    dn = db / (dnorm + 1e-8)
    s = lax.dot_general(dn, qn_ref[...], (((1,), (1,)), ((), ())),
                        preferred_element_type=jnp.float32)
    m = jnp.max(s, axis=0)
    cand = jnp.argmax(s, axis=0).astype(jnp.int32)
    gidx = cand + i * n_chunk
    better = m > val_ref[...]
    val_ref[...] = jnp.where(better, m, val_ref[...])
    idx_ref[...] = jnp.where(better, gidx, idx_ref[...])


def _topk_call(x, W, db):
    b, d = x.shape
    n = db.shape[0]
    chunk = _pick_chunk(n)
    nsteps = n // chunk
    return pl.pallas_call(
        functools.partial(_topk_body, chunk),
        grid=(nsteps,),
        in_specs=[
            pl.BlockSpec((b, d), lambda i: (0, 0)),
            pl.BlockSpec((d, d), lambda i: (0, 0)),
            pl.BlockSpec((chunk, d), lambda i: (i, 0)),
        ],
        out_specs=[
            pl.BlockSpec((b,), lambda i: (0,)),
            pl.BlockSpec((b,), lambda i: (0,)),
        ],
        out_shape=[
            jax.ShapeDtypeStruct((b,), jnp.float32),
            jax.ShapeDtypeStruct((b,), jnp.int32),
        ],
        scratch_shapes=[pltpu.VMEM((b, d), jnp.float32)],
    )(x, W, db)


def _label_gather(labels, idx):
    b = idx.shape[0]
    info = plsc.get_sparse_core_info()
    nw = info.num_cores * info.num_subcores
    bpw = b // nw
    mesh = plsc.VectorSubcoreMesh(core_axis_name="c", subcore_axis_name="s")

    @functools.partial(
        pl.kernel,
        mesh=mesh,
        out_type=jax.ShapeDtypeStruct((b,), jnp.int32),
        scratch_types=[
            pltpu.VMEM((bpw,), jnp.int32),
            pltpu.VMEM((bpw,), jnp.int32),
            pltpu.SemaphoreType.DMA,
        ],
    )
    def k(labels_hbm, idx_hbm, out_hbm, idx_v, vals_v, sem):
        wid = lax.axis_index("s") * info.num_cores + lax.axis_index("c")
        base = wid * bpw
        pltpu.sync_copy(idx_hbm.at[pl.ds(base, bpw)], idx_v)
        pltpu.async_copy(labels_hbm.at[idx_v], vals_v, sem).wait()
        pltpu.sync_copy(vals_v, out_hbm.at[pl.ds(base, bpw)])

    return k(labels, idx)


def kernel(x, W, db_features, db_labels):
    top_vals, top_idx = _topk_call(x, W, db_features)
    pred = _label_gather(db_labels, top_idx)
    return top_vals, top_idx, pred


# speedup vs baseline: 5.8739x; 1.0618x over previous
import functools

import jax
import jax.numpy as jnp
from jax import lax
from jax.experimental import pallas as pl
from jax.experimental.pallas import tpu as pltpu
from jax.experimental.pallas import tpu_sc as plsc


def _pick_chunk(n: int, cap: int = 4096) -> int:
    for c in range(min(n, cap), 7, -1):
        if n % c == 0 and c % 8 == 0:
            return c
    return n


def _topk_body(n_chunk, x_ref, w_ref, db_ref, val_ref, idx_ref, qn_ref):
    i = pl.program_id(0)

    @pl.when(i == 0)
    def _init():
        feats = jnp.dot(x_ref[...], w_ref[...],
                        preferred_element_type=jnp.float32)
        qnorm = jnp.sqrt(jnp.sum(feats * feats, axis=1, keepdims=True))
        qn_ref[...] = feats / (qnorm + 1e-8)
        val_ref[...] = jnp.full(val_ref.shape, -jnp.inf, jnp.float32)
        idx_ref[...] = jnp.zeros(idx_ref.shape, jnp.int32)

    db = db_ref[...]
    ones = jnp.ones((db.shape[1], 1), jnp.float32)
    dsq = jnp.dot(db * db, ones, preferred_element_type=jnp.float32)
    dn = db / (jnp.sqrt(dsq) + 1e-8)
    s = lax.dot_general(dn, qn_ref[...], (((1,), (1,)), ((), ())),
                        preferred_element_type=jnp.float32)
    m = jnp.max(s, axis=0)
    cand = jnp.argmax(s, axis=0).astype(jnp.int32)
    gidx = cand + i * n_chunk
    better = m > val_ref[...]
    val_ref[...] = jnp.where(better, m, val_ref[...])
    idx_ref[...] = jnp.where(better, gidx, idx_ref[...])


def _topk_call(x, W, db):
    b, d = x.shape
    n = db.shape[0]
    chunk = _pick_chunk(n)
    nsteps = n // chunk
    return pl.pallas_call(
        functools.partial(_topk_body, chunk),
        grid=(nsteps,),
        in_specs=[
            pl.BlockSpec((b, d), lambda i: (0, 0)),
            pl.BlockSpec((d, d), lambda i: (0, 0)),
            pl.BlockSpec((chunk, d), lambda i: (i, 0)),
        ],
        out_specs=[
            pl.BlockSpec((b,), lambda i: (0,)),
            pl.BlockSpec((b,), lambda i: (0,)),
        ],
        out_shape=[
            jax.ShapeDtypeStruct((b,), jnp.float32),
            jax.ShapeDtypeStruct((b,), jnp.int32),
        ],
        scratch_shapes=[pltpu.VMEM((b, d), jnp.float32)],
    )(x, W, db)


def _label_gather(labels, idx):
    b = idx.shape[0]
    info = plsc.get_sparse_core_info()
    nw = info.num_cores * info.num_subcores
    bpw = b // nw
    mesh = plsc.VectorSubcoreMesh(core_axis_name="c", subcore_axis_name="s")

    @functools.partial(
        pl.kernel,
        mesh=mesh,
        out_type=jax.ShapeDtypeStruct((b,), jnp.int32),
        scratch_types=[
            pltpu.VMEM((bpw,), jnp.int32),
            pltpu.VMEM((bpw,), jnp.int32),
            pltpu.SemaphoreType.DMA,
        ],
    )
    def k(labels_hbm, idx_hbm, out_hbm, idx_v, vals_v, sem):
        wid = lax.axis_index("s") * info.num_cores + lax.axis_index("c")
        base = wid * bpw
        pltpu.sync_copy(idx_hbm.at[pl.ds(base, bpw)], idx_v)
        pltpu.async_copy(labels_hbm.at[idx_v], vals_v, sem).wait()
        pltpu.sync_copy(vals_v, out_hbm.at[pl.ds(base, bpw)])

    return k(labels, idx)


def kernel(x, W, db_features, db_labels):
    top_vals, top_idx = _topk_call(x, W, db_features)
    pred = _label_gather(db_labels, top_idx)
    return top_vals, top_idx, pred


# speedup vs baseline: 5.8985x; 1.0042x over previous
import functools

import jax
import jax.numpy as jnp
from jax import lax
from jax.experimental import pallas as pl
from jax.experimental.pallas import tpu as pltpu
from jax.experimental.pallas import tpu_sc as plsc


def _pick_chunk(n: int, cap: int = 4096) -> int:
    for c in range(min(n, cap), 7, -1):
        if n % c == 0 and c % 8 == 0:
            return c
    return n


def _topk_body(n_chunk, x_ref, w_ref, db_ref, val_ref, idx_ref, qn_ref):
    i = pl.program_id(0)

    @pl.when(i == 0)
    def _init():
        feats = jnp.dot(x_ref[...], w_ref[...],
                        preferred_element_type=jnp.float32)
        qnorm = jnp.sqrt(jnp.sum(feats * feats, axis=1, keepdims=True))
        qn_ref[...] = feats / (qnorm + 1e-8)
        val_ref[...] = jnp.full(val_ref.shape, -jnp.inf, jnp.float32)
        idx_ref[...] = jnp.zeros(idx_ref.shape, jnp.int32)

    db = db_ref[...]
    dnorm = jnp.sqrt(jnp.sum(db * db, axis=1, keepdims=True))
    dn = db / (dnorm + 1e-8)
    s = lax.dot_general(dn, qn_ref[...], (((1,), (1,)), ((), ())),
                        preferred_element_type=jnp.float32)
    m = jnp.max(s, axis=0)
    cand = jnp.argmax(s, axis=0).astype(jnp.int32)
    gidx = cand + i * n_chunk
    better = m > val_ref[...]
    val_ref[...] = jnp.where(better, m, val_ref[...])
    idx_ref[...] = jnp.where(better, gidx, idx_ref[...])


def _topk_call(x, W, db):
    b, d = x.shape
    n = db.shape[0]
    chunk = _pick_chunk(n)
    nsteps = n // chunk
    return pl.pallas_call(
        functools.partial(_topk_body, chunk),
        grid=(nsteps,),
        in_specs=[
            pl.BlockSpec((b, d), lambda i: (0, 0)),
            pl.BlockSpec((d, d), lambda i: (0, 0)),
            pl.BlockSpec((chunk, d), lambda i: (i, 0)),
        ],
        out_specs=[
            pl.BlockSpec((b,), lambda i: (0,)),
            pl.BlockSpec((b,), lambda i: (0,)),
        ],
        out_shape=[
            jax.ShapeDtypeStruct((b,), jnp.float32),
            jax.ShapeDtypeStruct((b,), jnp.int32),
        ],
        scratch_shapes=[pltpu.VMEM((b, d), jnp.float32)],
    )(x, W, db)


def _label_gather(labels, idx):
    b = idx.shape[0]
    info = plsc.get_sparse_core_info()
    nw = info.num_cores * info.num_subcores
    bpw = b // nw
    mesh = plsc.VectorSubcoreMesh(core_axis_name="c", subcore_axis_name="s")

    @functools.partial(
        pl.kernel,
        mesh=mesh,
        out_type=jax.ShapeDtypeStruct((b,), jnp.int32),
        scratch_types=[
            pltpu.VMEM((bpw,), jnp.int32),
            pltpu.VMEM((bpw,), jnp.int32),
            pltpu.SemaphoreType.DMA,
        ],
    )
    def k(labels_hbm, idx_hbm, out_hbm, idx_v, vals_v, sem):
        wid = lax.axis_index("s") * info.num_cores + lax.axis_index("c")
        base = wid * bpw
        pltpu.sync_copy(idx_hbm.at[pl.ds(base, bpw)], idx_v)
        pltpu.async_copy(labels_hbm.at[idx_v], vals_v, sem).wait()
        pltpu.sync_copy(vals_v, out_hbm.at[pl.ds(base, bpw)])

    return k(labels, idx)


def kernel(x, W, db_features, db_labels):
    top_vals, top_idx = _topk_call(x, W, db_features)
    pred = _label_gather(db_labels, top_idx)
    return top_vals, top_idx, pred


# speedup vs baseline: 5.9327x; 1.0058x over previous
import functools

import jax
import jax.numpy as jnp
from jax import lax
from jax.experimental import pallas as pl
from jax.experimental.pallas import tpu as pltpu
from jax.experimental.pallas import tpu_sc as plsc


def _pick_chunk(n: int, cap: int = 5120) -> int:
    for c in range(min(n, cap), 7, -1):
        if n % c == 0 and c % 8 == 0:
            return c
    return n


def _topk_body(n_chunk, x_ref, w_ref, db_ref, val_ref, idx_ref, qn_ref):
    i = pl.program_id(0)

    @pl.when(i == 0)
    def _init():
        feats = jnp.dot(x_ref[...], w_ref[...],
                        preferred_element_type=jnp.float32)
        qnorm = jnp.sqrt(jnp.sum(feats * feats, axis=1, keepdims=True))
        qn_ref[...] = feats / (qnorm + 1e-8)
        val_ref[...] = jnp.full(val_ref.shape, -jnp.inf, jnp.float32)
        idx_ref[...] = jnp.zeros(idx_ref.shape, jnp.int32)

    db = db_ref[...]
    dnorm = jnp.sqrt(jnp.sum(db * db, axis=1, keepdims=True))
    dn = db / (dnorm + 1e-8)
    s = lax.dot_general(dn, qn_ref[...], (((1,), (1,)), ((), ())),
                        preferred_element_type=jnp.float32)
    m = jnp.max(s, axis=0)
    cand = jnp.argmax(s, axis=0).astype(jnp.int32)
    gidx = cand + i * n_chunk
    better = m > val_ref[...]
    val_ref[...] = jnp.where(better, m, val_ref[...])
    idx_ref[...] = jnp.where(better, gidx, idx_ref[...])


def _topk_call(x, W, db):
    b, d = x.shape
    n = db.shape[0]
    chunk = _pick_chunk(n)
    nsteps = n // chunk
    return pl.pallas_call(
        functools.partial(_topk_body, chunk),
        grid=(nsteps,),
        in_specs=[
            pl.BlockSpec((b, d), lambda i: (0, 0)),
            pl.BlockSpec((d, d), lambda i: (0, 0)),
            pl.BlockSpec((chunk, d), lambda i: (i, 0)),
        ],
        out_specs=[
            pl.BlockSpec((b,), lambda i: (0,)),
            pl.BlockSpec((b,), lambda i: (0,)),
        ],
        out_shape=[
            jax.ShapeDtypeStruct((b,), jnp.float32),
            jax.ShapeDtypeStruct((b,), jnp.int32),
        ],
        scratch_shapes=[pltpu.VMEM((b, d), jnp.float32)],
    )(x, W, db)


def _label_gather(labels, idx):
    b = idx.shape[0]
    info = plsc.get_sparse_core_info()
    nw = info.num_cores * info.num_subcores
    bpw = b // nw
    mesh = plsc.VectorSubcoreMesh(core_axis_name="c", subcore_axis_name="s")

    @functools.partial(
        pl.kernel,
        mesh=mesh,
        out_type=jax.ShapeDtypeStruct((b,), jnp.int32),
        scratch_types=[
            pltpu.VMEM((bpw,), jnp.int32),
            pltpu.VMEM((bpw,), jnp.int32),
            pltpu.SemaphoreType.DMA,
        ],
    )
    def k(labels_hbm, idx_hbm, out_hbm, idx_v, vals_v, sem):
        wid = lax.axis_index("s") * info.num_cores + lax.axis_index("c")
        base = wid * bpw
        pltpu.sync_copy(idx_hbm.at[pl.ds(base, bpw)], idx_v)
        pltpu.async_copy(labels_hbm.at[idx_v], vals_v, sem).wait()
        pltpu.sync_copy(vals_v, out_hbm.at[pl.ds(base, bpw)])

    return k(labels, idx)


def kernel(x, W, db_features, db_labels):
    top_vals, top_idx = _topk_call(x, W, db_features)
    pred = _label_gather(db_labels, top_idx)
    return top_vals, top_idx, pred
